# baseline (speedup 1.0000x reference)
import functools

import jax
import jax.numpy as jnp
import numpy as np
from jax import lax
from jax.experimental import pallas as pl
from jax.experimental.pallas import tpu as pltpu
from jax.experimental.pallas import tpu_sc as plsc

N_NODES_REAL = 10000
D_IN = 128
D_OUT = 64
N_EDGES_REAL = 320000

NC, NS = 2, 16
NW = NC * NS
CH = 128
NBUF = 2
N_PAD = 10240
JUNK = N_NODES_REAL
N_IT = 80
E_PAD = NW * CH * N_IT
EPT = CH * N_IT
ROWS_PER_SC_TILE = N_PAD // NS
INIT_CHUNKS = ROWS_PER_SC_TILE // CH

_MULT_I32 = (
    np.random.default_rng(0)
    .integers(1, 2**31 - 1, size=(D_IN,))
    .astype(np.int64)
    .astype(np.int32)
    .reshape(1, D_IN)
)

_mesh = plsc.VectorSubcoreMesh(core_axis_name="c", subcore_axis_name="s")


@functools.partial(
    pl.kernel,
    mesh=_mesh,
    out_type=jax.ShapeDtypeStruct((NC * N_PAD, D_IN), jnp.float32),
    scratch_types=[
        pltpu.VMEM((N_IT, CH), jnp.int32),
        pltpu.VMEM((CH, D_IN), jnp.float32),
        pltpu.VMEM_SHARED((N_PAD, D_IN), jnp.float32),
        pltpu.SemaphoreType.DMA,
        pltpu.SemaphoreType.DMA,
    ],
)
def _sc_degree(dst_hbm, ones_hbm, out_hbm, idx_v, ones_v, acc, sem, sem_w):
    c = lax.axis_index("c")
    s = lax.axis_index("s")
    wid = s * NC + c
    pltpu.sync_copy(ones_hbm, ones_v)
    h_idx = pltpu.async_copy(dst_hbm.at[wid], idx_v, sem_w)
    inits = [None] * INIT_CHUNKS
    for k in range(INIT_CHUNKS):
        row0 = s * ROWS_PER_SC_TILE + k * CH
        inits[k] = pltpu.async_copy(ones_v, acc.at[pl.ds(row0, CH)], sem)
    for k in range(INIT_CHUNKS):
        inits[k].wait()
    h_idx.wait()
    plsc.subcore_barrier()

    def body(i, carry):
        hs = [None] * 8
        for b in range(8):
            hs[b] = pltpu.async_copy(
                ones_v, acc.at[idx_v.at[i * 8 + b]], sem, add=True)
        for b in range(8):
            hs[b].wait()
        return carry

    lax.fori_loop(0, N_IT // 8, body, 0)
    plsc.subcore_barrier()
    ws = [None] * INIT_CHUNKS
    for k in range(INIT_CHUNKS):
        row0 = s * ROWS_PER_SC_TILE + k * CH
        ws[k] = pltpu.async_copy(
            acc.at[pl.ds(row0, CH)],
            out_hbm.at[pl.ds(c * N_PAD + row0, CH)], sem_w)
    for k in range(INIT_CHUNKS):
        ws[k].wait()


ECH = 64
ENIT = EPT // ECH
EHALF = ENIT // 4
ENBUF = 4


@functools.partial(
    pl.kernel,
    mesh=_mesh,
    out_type=jax.ShapeDtypeStruct((NC * N_PAD, D_IN), jnp.float32),
    scratch_types=[
        pltpu.VMEM((EHALF, ECH), jnp.int32),
        pltpu.VMEM((EHALF, ECH), jnp.int32),
        pltpu.VMEM((ENBUF, ECH, D_IN), jnp.float32),
        pltpu.VMEM_SHARED((N_PAD, D_IN), jnp.float32),
        pltpu.SemaphoreType.DMA,
        pltpu.SemaphoreType.DMA,
        pltpu.SemaphoreType.DMA,
        pltpu.SemaphoreType.DMA,
        pltpu.SemaphoreType.DMA,
        pltpu.SemaphoreType.DMA,
        pltpu.SemaphoreType.DMA,
        pltpu.SemaphoreType.DMA,
        pltpu.SemaphoreType.DMA,
    ],
)
def _sc_hop(y_hbm, src_hbm, dst_hbm, zeros_hbm, out_hbm,
            src_v, dst_v, rows_v, acc,
            sg0, sg1, sg2, sg3, ss0, ss1, ss2, ss3, sem_w):
    c = lax.axis_index("c")
    s = lax.axis_index("s")
    wid = s * NC + c
    sg = [sg0, sg1, sg2, sg3]
    ss = [ss0, ss1, ss2, ss3]
    for b in range(ENBUF):
        pltpu.sync_copy(zeros_hbm.at[pl.ds(0, ECH)], rows_v.at[b])
    nz = ROWS_PER_SC_TILE // ECH
    zs = [None] * nz
    for k in range(nz):
        row0 = s * ROWS_PER_SC_TILE + k * ECH
        zs[k] = pltpu.async_copy(
            rows_v.at[k % ENBUF], acc.at[pl.ds(row0, ECH)], sem_w)
    for k in range(nz):
        zs[k].wait()
    plsc.subcore_barrier()

    def body(i, carry):
        g0 = i * ENBUF
        hg = [None] * ENBUF
        hs = [None] * ENBUF
        for b in range(ENBUF):
            hg[b] = pltpu.async_copy(
                y_hbm.at[c].at[src_v.at[g0 + b]], rows_v.at[b], sg[b])
        for b in range(ENBUF):
            hg[b].wait()
            hs[b] = pltpu.async_copy(
                rows_v.at[b], acc.at[dst_v.at[g0 + b]], ss[b], add=True)
        for b in range(ENBUF):
            hs[b].wait()
        return carry

    for h in range(ENIT // EHALF):
        pltpu.sync_copy(src_hbm.at[wid, pl.ds(h * EHALF, EHALF)], src_v)
        pltpu.sync_copy(dst_hbm.at[wid, pl.ds(h * EHALF, EHALF)], dst_v)
        lax.fori_loop(0, EHALF // ENBUF, body, 0)
    plsc.subcore_barrier()
    ws = [None] * INIT_CHUNKS
    for k in range(INIT_CHUNKS):
        row0 = s * ROWS_PER_SC_TILE + k * CH
        ws[k] = pltpu.async_copy(
            acc.at[pl.ds(row0, CH)],
            out_hbm.at[pl.ds(c * N_PAD + row0, CH)], sem_w)
    for k in range(INIT_CHUNKS):
        ws[k].wait()


RPT = N_PAD // NW


@functools.partial(
    pl.kernel,
    mesh=_mesh,
    out_type=jax.ShapeDtypeStruct((NW, RPT, D_IN), jnp.float32),
    scratch_types=[
        pltpu.VMEM((RPT,), jnp.int32),
        pltpu.VMEM((RPT, D_IN), jnp.float32),
        pltpu.VMEM_SHARED((N_PAD, D_IN), jnp.float32),
        pltpu.SemaphoreType.DMA,
        pltpu.SemaphoreType.DMA,
    ],
)
def _sc_gather_rows(ls_hbm, rep_hbm, out_hbm, idx_v, rows_v, ls_spm,
                    sem, sem_w):
    c = lax.axis_index("c")
    s = lax.axis_index("s")
    wid = s * NC + c
    for r in range(ROWS_PER_SC_TILE // RPT):
        row0 = s * ROWS_PER_SC_TILE + r * RPT
        pltpu.sync_copy(ls_hbm.at[pl.ds(row0, RPT)], rows_v)
        pltpu.sync_copy(rows_v, ls_spm.at[pl.ds(row0, RPT)])
    pltpu.sync_copy(rep_hbm.at[wid], idx_v)
    plsc.subcore_barrier()
    h1 = pltpu.async_copy(ls_spm.at[idx_v.at[pl.ds(0, CH)]],
                          rows_v.at[pl.ds(0, CH)], sem)
    h2 = pltpu.async_copy(ls_spm.at[idx_v.at[pl.ds(CH, CH)]],
                          rows_v.at[pl.ds(CH, CH)], sem)
    h3 = pltpu.async_copy(ls_spm.at[idx_v.at[pl.ds(2 * CH, RPT - 2 * CH)]],
                          rows_v.at[pl.ds(2 * CH, RPT - 2 * CH)], sem)
    h1.wait()
    h2.wait()
    h3.wait()
    pltpu.sync_copy(rows_v, out_hbm.at[wid])


def _tc_prep(degp, x_pad):
    def body(degp_ref, x_ref, dinv_ref, y0_ref):
        d = degp_ref[0, :, 0:1] + degp_ref[1, :, 0:1] - 1.0
        dinv = lax.rsqrt(jnp.maximum(d, 1.0))
        dinv_ref[...] = dinv
        y0 = x_ref[...] * dinv
        y0_ref[0] = y0
        y0_ref[1] = y0

    return pl.pallas_call(
        body,
        out_shape=(
            jax.ShapeDtypeStruct((N_PAD, 1), jnp.float32),
            jax.ShapeDtypeStruct((2, N_PAD, D_IN), jnp.float32),
        ),
    )(degp, x_pad)


def _tc_scale2(sp, y0, dinv):
    def body(sp_ref, y0_ref, dinv_ref, y1_ref):
        dv = dinv_ref[...]
        y1 = (y0_ref[0] + sp_ref[0] + sp_ref[1]) * (dv * dv)
        y1_ref[0] = y1
        y1_ref[1] = y1

    return pl.pallas_call(
        body,
        out_shape=jax.ShapeDtypeStruct((2, N_PAD, D_IN), jnp.float32),
    )(sp, y0, dinv)


def _tc_mlp(sp2, y1, dinv, W1, b1, W2, b2, mult):
    def body(sp_ref, y1_ref, dinv_ref, w1_ref, b1_ref, w2_ref, b2_ref, m_ref,
             ls_ref, h_ref):
        xp = (y1_ref[0] + sp_ref[0] + sp_ref[1]) * dinv_ref[...]
        keys = jnp.round(xp).astype(jnp.int32)
        h_ref[...] = jnp.sum(keys * m_ref[...], axis=1, keepdims=True,
                             dtype=jnp.int32)
        hid = jnp.maximum(
            lax.dot_general(xp, w1_ref[...], (((1,), (1,)), ((), ())),
                            preferred_element_type=jnp.float32)
            + b1_ref[...], 0.0)
        o = lax.dot_general(hid, w2_ref[...], (((1,), (1,)), ((), ())),
                            preferred_element_type=jnp.float32) + b2_ref[...]
        o = o - jnp.max(o, axis=1, keepdims=True)
        ls = o - jnp.log(jnp.sum(jnp.exp(o), axis=1, keepdims=True))
        ls_ref[...] = jnp.concatenate([ls, jnp.zeros_like(ls)], axis=1)

    return pl.pallas_call(
        body,
        out_shape=(
            jax.ShapeDtypeStruct((N_PAD, D_IN), jnp.float32),
            jax.ShapeDtypeStruct((N_PAD, 1), jnp.int32),
        ),
    )(sp2, y1, dinv, W1, b1.reshape(1, D_IN), W2, b2.reshape(1, D_OUT), mult)


REP_BI = 1024
REP_BJ = 512


def _tc_rep(h_col, h_row):
    BIG = 2**30

    def body(hA_ref, hB_ref, rep_ref, best_ref):
        hi = hA_ref[...]
        best_ref[...] = jnp.full((REP_BI, 1), BIG, jnp.int32)
        blk = pl.program_id(0)
        for k in range(N_PAD // REP_BJ):
            @pl.when(k * REP_BJ < (blk + 1) * REP_BI)
            def _():
                hj = hB_ref[:, k * REP_BJ:(k + 1) * REP_BJ]
                eq = hi == hj
                jidx = lax.broadcasted_iota(
                    jnp.int32, (REP_BI, REP_BJ), 1) + jnp.int32(k * REP_BJ)
                cand = jnp.where(eq, jidx, BIG)
                best_ref[...] = jnp.minimum(
                    best_ref[...], jnp.min(cand, axis=1, keepdims=True))

        rep_ref[...] = best_ref[...]

    return pl.pallas_call(
        body,
        grid=(N_PAD // REP_BI,),
        in_specs=[
            pl.BlockSpec((REP_BI, 1), lambda i: (i, 0)),
            pl.BlockSpec((1, N_PAD), lambda i: (0, 0)),
        ],
        out_specs=pl.BlockSpec((REP_BI, 1), lambda i: (i, 0)),
        out_shape=jax.ShapeDtypeStruct((N_PAD, 1), jnp.int32),
        scratch_shapes=[pltpu.VMEM((REP_BI, 1), jnp.int32)],
    )(h_col, h_row)


def kernel(x, edge_index, W1, b1, W2, b2):
    x = x.astype(jnp.float32)
    ei = edge_index.astype(jnp.int32)
    n_extra = E_PAD - N_EDGES_REAL
    src = jnp.concatenate([ei[0], jnp.zeros((n_extra,), jnp.int32)])
    dst = jnp.concatenate([ei[1], jnp.full((n_extra,), JUNK, jnp.int32)])
    src3 = src.reshape(NW, N_IT, CH)
    dst3 = dst.reshape(NW, N_IT, CH)
    x_pad = jnp.pad(x, ((0, N_PAD - N_NODES_REAL), (0, 0)))
    ones128 = jnp.ones((CH, D_IN), jnp.float32)
    zeros128 = jnp.zeros((CH, D_IN), jnp.float32)

    src3h = src.reshape(NW, ENIT, ECH)
    dst3h = dst.reshape(NW, ENIT, ECH)

    degp = _sc_degree(dst3, ones128).reshape(2, N_PAD, D_IN)
    dinv, y0 = _tc_prep(degp, x_pad)
    sp1 = _sc_hop(y0, src3h, dst3h, zeros128).reshape(2, N_PAD, D_IN)
    y1 = _tc_scale2(sp1, y0, dinv)
    sp2 = _sc_hop(y1, src3h, dst3h, zeros128).reshape(2, N_PAD, D_IN)
    ls, h = _tc_mlp(sp2, y1, dinv, W1, b1, W2, b2,
                    jnp.asarray(_MULT_I32, dtype=jnp.int32))
    rep = _tc_rep(h, h.reshape(1, N_PAD))
    out = _sc_gather_rows(ls, rep.reshape(NW, RPT)).reshape(N_PAD, D_IN)
    return out[:N_NODES_REAL, :D_OUT]


# speedup vs baseline: 1.1850x; 1.1850x over previous
import functools

import jax
import jax.numpy as jnp
import numpy as np
from jax import lax
from jax.experimental import pallas as pl
from jax.experimental.pallas import tpu as pltpu
from jax.experimental.pallas import tpu_sc as plsc

N_NODES_REAL = 10000
D_IN = 128
D_OUT = 64
N_EDGES_REAL = 320000

NC, NS = 2, 16
NW = NC * NS
CH = 128
NBUF = 2
N_PAD = 10240
JUNK = N_NODES_REAL
N_IT = 80
E_PAD = NW * CH * N_IT
EPT = CH * N_IT
ROWS_PER_SC_TILE = N_PAD // NS
INIT_CHUNKS = ROWS_PER_SC_TILE // CH

_MULT_I32 = (
    np.random.default_rng(0)
    .integers(1, 2**31 - 1, size=(D_IN,))
    .astype(np.int64)
    .astype(np.int32)
    .reshape(1, D_IN)
)

_mesh = plsc.VectorSubcoreMesh(core_axis_name="c", subcore_axis_name="s")


@functools.partial(
    pl.kernel,
    mesh=_mesh,
    out_type=jax.ShapeDtypeStruct((NC * N_PAD, D_IN), jnp.float32),
    scratch_types=[
        pltpu.VMEM((N_IT, CH), jnp.int32),
        pltpu.VMEM((CH, D_IN), jnp.float32),
        pltpu.VMEM_SHARED((N_PAD, D_IN), jnp.float32),
        pltpu.SemaphoreType.DMA,
        pltpu.SemaphoreType.DMA,
    ],
)
def _sc_degree(dst_hbm, ones_hbm, out_hbm, idx_v, ones_v, acc, sem, sem_w):
    c = lax.axis_index("c")
    s = lax.axis_index("s")
    wid = s * NC + c
    pltpu.sync_copy(ones_hbm, ones_v)
    h_idx = pltpu.async_copy(dst_hbm.at[wid], idx_v, sem_w)
    inits = [None] * INIT_CHUNKS
    for k in range(INIT_CHUNKS):
        row0 = s * ROWS_PER_SC_TILE + k * CH
        inits[k] = pltpu.async_copy(ones_v, acc.at[pl.ds(row0, CH)], sem)
    for k in range(INIT_CHUNKS):
        inits[k].wait()
    h_idx.wait()
    plsc.subcore_barrier()

    def body(i, carry):
        hs = [None] * 8
        for b in range(8):
            hs[b] = pltpu.async_copy(
                ones_v, acc.at[idx_v.at[i * 8 + b]], sem, add=True)
        for b in range(8):
            hs[b].wait()
        return carry

    lax.fori_loop(0, N_IT // 8, body, 0)
    plsc.subcore_barrier()
    ws = [None] * INIT_CHUNKS
    for k in range(INIT_CHUNKS):
        row0 = s * ROWS_PER_SC_TILE + k * CH
        ws[k] = pltpu.async_copy(
            acc.at[pl.ds(row0, CH)],
            out_hbm.at[pl.ds(c * N_PAD + row0, CH)], sem_w)
    for k in range(INIT_CHUNKS):
        ws[k].wait()


ECH = 64
NCHUNKS = E_PAD // ECH
EHALF = 40
ENBUF = 4
LOADS_SC0 = 6
LOADS_SC1 = NCHUNKS // (16 * EHALF) - LOADS_SC0


@functools.partial(
    pl.kernel,
    mesh=_mesh,
    out_type=jax.ShapeDtypeStruct((NC * N_PAD, D_IN), jnp.float32),
    scratch_types=[
        pltpu.VMEM((EHALF, ECH), jnp.int32),
        pltpu.VMEM((EHALF, ECH), jnp.int32),
        pltpu.VMEM((ENBUF, ECH, D_IN), jnp.float32),
        pltpu.VMEM_SHARED((N_PAD, D_IN), jnp.float32),
        pltpu.SemaphoreType.DMA,
        pltpu.SemaphoreType.DMA,
        pltpu.SemaphoreType.DMA,
        pltpu.SemaphoreType.DMA,
        pltpu.SemaphoreType.DMA,
        pltpu.SemaphoreType.DMA,
        pltpu.SemaphoreType.DMA,
        pltpu.SemaphoreType.DMA,
        pltpu.SemaphoreType.DMA,
    ],
)
def _sc_hop(y_hbm, src_hbm, dst_hbm, zeros_hbm, out_hbm,
            src_v, dst_v, rows_v, acc,
            sg0, sg1, sg2, sg3, ss0, ss1, ss2, ss3, sem_w):
    c = lax.axis_index("c")
    s = lax.axis_index("s")
    wid = s * NC + c
    sg = [sg0, sg1, sg2, sg3]
    ss = [ss0, ss1, ss2, ss3]
    for b in range(ENBUF):
        pltpu.sync_copy(zeros_hbm.at[pl.ds(0, ECH)], rows_v.at[b])
    nz = ROWS_PER_SC_TILE // ECH
    zs = [None] * nz
    for k in range(nz):
        row0 = s * ROWS_PER_SC_TILE + k * ECH
        zs[k] = pltpu.async_copy(
            rows_v.at[k % ENBUF], acc.at[pl.ds(row0, ECH)], sem_w)
    for k in range(nz):
        zs[k].wait()
    plsc.subcore_barrier()

    def body(i, carry):
        g0 = i * ENBUF
        hg = [None] * ENBUF
        hs = [None] * ENBUF
        for b in range(ENBUF):
            hg[b] = pltpu.async_copy(
                y_hbm.at[src_v.at[g0 + b]], rows_v.at[b], sg[b])
        for b in range(ENBUF):
            hg[b].wait()
            hs[b] = pltpu.async_copy(
                rows_v.at[b], acc.at[dst_v.at[g0 + b]], ss[b], add=True)
        for b in range(ENBUF):
            hs[b].wait()
        return carry

    grp0 = jnp.where(c == 0, s * LOADS_SC0,
                     16 * LOADS_SC0 + s * LOADS_SC1)

    for h in range(LOADS_SC0):
        def _run(h=h):
            base = (grp0 + h) * EHALF
            pltpu.sync_copy(src_hbm.at[pl.ds(base, EHALF)], src_v)
            pltpu.sync_copy(dst_hbm.at[pl.ds(base, EHALF)], dst_v)
            lax.fori_loop(0, EHALF // ENBUF, body, 0)

        if h < LOADS_SC1:
            _run()
        else:
            pl.when(c == 0)(_run)
    plsc.subcore_barrier()
    ws = [None] * INIT_CHUNKS
    for k in range(INIT_CHUNKS):
        row0 = s * ROWS_PER_SC_TILE + k * CH
        ws[k] = pltpu.async_copy(
            acc.at[pl.ds(row0, CH)],
            out_hbm.at[pl.ds(c * N_PAD + row0, CH)], sem_w)
    for k in range(INIT_CHUNKS):
        ws[k].wait()


RPT = N_PAD // NW


@functools.partial(
    pl.kernel,
    mesh=_mesh,
    out_type=jax.ShapeDtypeStruct((NW, RPT, D_IN), jnp.float32),
    scratch_types=[
        pltpu.VMEM((RPT,), jnp.int32),
        pltpu.VMEM((RPT, D_IN), jnp.float32),
        pltpu.VMEM_SHARED((N_PAD, D_IN), jnp.float32),
        pltpu.SemaphoreType.DMA,
        pltpu.SemaphoreType.DMA,
    ],
)
def _sc_gather_rows(ls_hbm, rep_hbm, out_hbm, idx_v, rows_v, ls_spm,
                    sem, sem_w):
    c = lax.axis_index("c")
    s = lax.axis_index("s")
    wid = s * NC + c
    for r in range(ROWS_PER_SC_TILE // RPT):
        row0 = s * ROWS_PER_SC_TILE + r * RPT
        pltpu.sync_copy(ls_hbm.at[pl.ds(row0, RPT)], rows_v)
        pltpu.sync_copy(rows_v, ls_spm.at[pl.ds(row0, RPT)])
    pltpu.sync_copy(rep_hbm.at[wid], idx_v)
    plsc.subcore_barrier()
    h1 = pltpu.async_copy(ls_spm.at[idx_v.at[pl.ds(0, CH)]],
                          rows_v.at[pl.ds(0, CH)], sem)
    h2 = pltpu.async_copy(ls_spm.at[idx_v.at[pl.ds(CH, CH)]],
                          rows_v.at[pl.ds(CH, CH)], sem)
    h3 = pltpu.async_copy(ls_spm.at[idx_v.at[pl.ds(2 * CH, RPT - 2 * CH)]],
                          rows_v.at[pl.ds(2 * CH, RPT - 2 * CH)], sem)
    h1.wait()
    h2.wait()
    h3.wait()
    pltpu.sync_copy(rows_v, out_hbm.at[wid])


def _tc_prep(degp, x_pad):
    def body(degp_ref, x_ref, dinv_ref, y0_ref):
        d = degp_ref[0, :, 0:1] + degp_ref[1, :, 0:1] - 1.0
        dinv = lax.rsqrt(jnp.maximum(d, 1.0))
        dinv_ref[...] = dinv
        y0_ref[...] = x_ref[...] * dinv

    return pl.pallas_call(
        body,
        out_shape=(
            jax.ShapeDtypeStruct((N_PAD, 1), jnp.float32),
            jax.ShapeDtypeStruct((N_PAD, D_IN), jnp.float32),
        ),
    )(degp, x_pad)


def _tc_scale2(sp, y0, dinv):
    def body(sp_ref, y0_ref, dinv_ref, y1_ref):
        dv = dinv_ref[...]
        y1_ref[...] = (y0_ref[...] + sp_ref[0] + sp_ref[1]) * (dv * dv)

    return pl.pallas_call(
        body,
        out_shape=jax.ShapeDtypeStruct((N_PAD, D_IN), jnp.float32),
    )(sp, y0, dinv)


def _tc_mlp(sp2, y1, dinv, W1, b1, W2, b2, mult):
    def body(sp_ref, y1_ref, dinv_ref, w1_ref, b1_ref, w2_ref, b2_ref, m_ref,
             ls_ref, h_ref):
        xp = (y1_ref[...] + sp_ref[0] + sp_ref[1]) * dinv_ref[...]
        keys = jnp.round(xp).astype(jnp.int32)
        h_ref[...] = jnp.sum(keys * m_ref[...], axis=1, keepdims=True,
                             dtype=jnp.int32)
        hid = jnp.maximum(
            lax.dot_general(xp, w1_ref[...], (((1,), (1,)), ((), ())),
                            preferred_element_type=jnp.float32)
            + b1_ref[...], 0.0)
        o = lax.dot_general(hid, w2_ref[...], (((1,), (1,)), ((), ())),
                            preferred_element_type=jnp.float32) + b2_ref[...]
        o = o - jnp.max(o, axis=1, keepdims=True)
        ls = o - jnp.log(jnp.sum(jnp.exp(o), axis=1, keepdims=True))
        ls_ref[...] = jnp.concatenate([ls, jnp.zeros_like(ls)], axis=1)

    return pl.pallas_call(
        body,
        out_shape=(
            jax.ShapeDtypeStruct((N_PAD, D_IN), jnp.float32),
            jax.ShapeDtypeStruct((N_PAD, 1), jnp.int32),
        ),
    )(sp2, y1, dinv, W1, b1.reshape(1, D_IN), W2, b2.reshape(1, D_OUT), mult)


REP_BI = 1024
REP_BJ = 512


def _tc_rep(h_col, h_row):
    BIG = 2**30

    def body(hA_ref, hB_ref, rep_ref, best_ref):
        hi = hA_ref[...]
        best_ref[...] = jnp.full((REP_BI, 1), BIG, jnp.int32)
        blk = pl.program_id(0)
        for k in range(N_PAD // REP_BJ):
            @pl.when(k * REP_BJ < (blk + 1) * REP_BI)
            def _():
                hj = hB_ref[:, k * REP_BJ:(k + 1) * REP_BJ]
                eq = hi == hj
                jidx = lax.broadcasted_iota(
                    jnp.int32, (REP_BI, REP_BJ), 1) + jnp.int32(k * REP_BJ)
                cand = jnp.where(eq, jidx, BIG)
                best_ref[...] = jnp.minimum(
                    best_ref[...], jnp.min(cand, axis=1, keepdims=True))

        rep_ref[...] = best_ref[...]

    return pl.pallas_call(
        body,
        grid=(N_PAD // REP_BI,),
        in_specs=[
            pl.BlockSpec((REP_BI, 1), lambda i: (i, 0)),
            pl.BlockSpec((1, N_PAD), lambda i: (0, 0)),
        ],
        out_specs=pl.BlockSpec((REP_BI, 1), lambda i: (i, 0)),
        out_shape=jax.ShapeDtypeStruct((N_PAD, 1), jnp.int32),
        scratch_shapes=[pltpu.VMEM((REP_BI, 1), jnp.int32)],
    )(h_col, h_row)


def kernel(x, edge_index, W1, b1, W2, b2):
    x = x.astype(jnp.float32)
    ei = edge_index.astype(jnp.int32)
    n_extra = E_PAD - N_EDGES_REAL
    src = jnp.concatenate([ei[0], jnp.zeros((n_extra,), jnp.int32)])
    dst = jnp.concatenate([ei[1], jnp.full((n_extra,), JUNK, jnp.int32)])
    src3 = src.reshape(NW, N_IT, CH)
    dst3 = dst.reshape(NW, N_IT, CH)
    x_pad = jnp.pad(x, ((0, N_PAD - N_NODES_REAL), (0, 0)))
    ones128 = jnp.ones((CH, D_IN), jnp.float32)
    zeros128 = jnp.zeros((CH, D_IN), jnp.float32)

    src2h = src.reshape(NCHUNKS, ECH)
    dst2h = dst.reshape(NCHUNKS, ECH)

    degp = _sc_degree(dst3, ones128).reshape(2, N_PAD, D_IN)
    dinv, y0 = _tc_prep(degp, x_pad)
    sp1 = _sc_hop(y0, src2h, dst2h, zeros128).reshape(2, N_PAD, D_IN)
    y1 = _tc_scale2(sp1, y0, dinv)
    sp2 = _sc_hop(y1, src2h, dst2h, zeros128).reshape(2, N_PAD, D_IN)
    ls, h = _tc_mlp(sp2, y1, dinv, W1, b1, W2, b2,
                    jnp.asarray(_MULT_I32, dtype=jnp.int32))
    rep = _tc_rep(h, h.reshape(1, N_PAD))
    out = _sc_gather_rows(ls, rep.reshape(NW, RPT)).reshape(N_PAD, D_IN)
    return out[:N_NODES_REAL, :D_OUT]


# speedup vs baseline: 1.2268x; 1.0353x over previous
import functools

import jax
import jax.numpy as jnp
import numpy as np
from jax import lax
from jax.experimental import pallas as pl
from jax.experimental.pallas import tpu as pltpu
from jax.experimental.pallas import tpu_sc as plsc

N_NODES_REAL = 10000
D_IN = 128
D_OUT = 64
N_EDGES_REAL = 320000

NC, NS = 2, 16
NW = NC * NS
CH = 128
NBUF = 2
N_PAD = 10240
JUNK = N_NODES_REAL
N_IT = 80
E_PAD = NW * CH * N_IT
EPT = CH * N_IT
ROWS_PER_SC_TILE = N_PAD // NS
INIT_CHUNKS = ROWS_PER_SC_TILE // CH

_MULT_I32 = (
    np.random.default_rng(0)
    .integers(1, 2**31 - 1, size=(D_IN,))
    .astype(np.int64)
    .astype(np.int32)
    .reshape(1, D_IN)
)

_mesh = plsc.VectorSubcoreMesh(core_axis_name="c", subcore_axis_name="s")


@functools.partial(
    pl.kernel,
    mesh=_mesh,
    out_type=jax.ShapeDtypeStruct((NC * N_PAD, D_IN), jnp.float32),
    scratch_types=[
        pltpu.VMEM((N_IT, CH), jnp.int32),
        pltpu.VMEM((CH, D_IN), jnp.float32),
        pltpu.VMEM_SHARED((N_PAD, D_IN), jnp.float32),
        pltpu.SemaphoreType.DMA,
        pltpu.SemaphoreType.DMA,
    ],
)
def _sc_degree(dst_hbm, ones_hbm, out_hbm, idx_v, ones_v, acc, sem, sem_w):
    c = lax.axis_index("c")
    s = lax.axis_index("s")
    wid = s * NC + c
    pltpu.sync_copy(ones_hbm, ones_v)
    h_idx = pltpu.async_copy(dst_hbm.at[wid], idx_v, sem_w)
    inits = [None] * INIT_CHUNKS
    for k in range(INIT_CHUNKS):
        row0 = s * ROWS_PER_SC_TILE + k * CH
        inits[k] = pltpu.async_copy(ones_v, acc.at[pl.ds(row0, CH)], sem)
    for k in range(INIT_CHUNKS):
        inits[k].wait()
    h_idx.wait()
    plsc.subcore_barrier()

    def body(i, carry):
        hs = [None] * 8
        for b in range(8):
            hs[b] = pltpu.async_copy(
                ones_v, acc.at[idx_v.at[i * 8 + b]], sem, add=True)
        for b in range(8):
            hs[b].wait()
        return carry

    lax.fori_loop(0, N_IT // 8, body, 0)
    plsc.subcore_barrier()
    ws = [None] * INIT_CHUNKS
    for k in range(INIT_CHUNKS):
        row0 = s * ROWS_PER_SC_TILE + k * CH
        ws[k] = pltpu.async_copy(
            acc.at[pl.ds(row0, CH)],
            out_hbm.at[pl.ds(c * N_PAD + row0, CH)], sem_w)
    for k in range(INIT_CHUNKS):
        ws[k].wait()


ECH = 64
NCHUNKS = E_PAD // ECH
EHALF = 40
ENBUF = 4
LOADS_SC0 = 7
LOADS_SC1 = NCHUNKS // (16 * EHALF) - LOADS_SC0


@functools.partial(
    pl.kernel,
    mesh=_mesh,
    out_type=jax.ShapeDtypeStruct((NC * N_PAD, D_IN), jnp.float32),
    scratch_types=[
        pltpu.VMEM((EHALF, ECH), jnp.int32),
        pltpu.VMEM((EHALF, ECH), jnp.int32),
        pltpu.VMEM((ENBUF, ECH, D_IN), jnp.float32),
        pltpu.VMEM_SHARED((N_PAD, D_IN), jnp.float32),
        pltpu.SemaphoreType.DMA,
        pltpu.SemaphoreType.DMA,
        pltpu.SemaphoreType.DMA,
        pltpu.SemaphoreType.DMA,
        pltpu.SemaphoreType.DMA,
        pltpu.SemaphoreType.DMA,
        pltpu.SemaphoreType.DMA,
        pltpu.SemaphoreType.DMA,
        pltpu.SemaphoreType.DMA,
    ],
)
def _sc_hop(y_hbm, src_hbm, dst_hbm, zeros_hbm, out_hbm,
            src_v, dst_v, rows_v, acc,
            sg0, sg1, sg2, sg3, ss0, ss1, ss2, ss3, sem_w):
    c = lax.axis_index("c")
    s = lax.axis_index("s")
    wid = s * NC + c
    sg = [sg0, sg1, sg2, sg3]
    ss = [ss0, ss1, ss2, ss3]
    for b in range(ENBUF):
        pltpu.sync_copy(zeros_hbm.at[pl.ds(0, ECH)], rows_v.at[b])
    nz = ROWS_PER_SC_TILE // ECH
    zs = [None] * nz
    for k in range(nz):
        row0 = s * ROWS_PER_SC_TILE + k * ECH
        zs[k] = pltpu.async_copy(
            rows_v.at[k % ENBUF], acc.at[pl.ds(row0, ECH)], sem_w)
    for k in range(nz):
        zs[k].wait()
    plsc.subcore_barrier()

    def body(i, carry):
        g0 = i * ENBUF
        hg = [None] * ENBUF
        hs = [None] * ENBUF
        for b in range(ENBUF):
            hg[b] = pltpu.async_copy(
                y_hbm.at[src_v.at[g0 + b]], rows_v.at[b], sg[b])
        for b in range(ENBUF):
            hg[b].wait()
            hs[b] = pltpu.async_copy(
                rows_v.at[b], acc.at[dst_v.at[g0 + b]], ss[b], add=True)
        for b in range(ENBUF):
            hs[b].wait()
        return carry

    grp0 = jnp.where(c == 0, s * LOADS_SC0,
                     16 * LOADS_SC0 + s * LOADS_SC1)

    for h in range(LOADS_SC0):
        def _run(h=h):
            base = (grp0 + h) * EHALF
            pltpu.sync_copy(src_hbm.at[pl.ds(base, EHALF)], src_v)
            pltpu.sync_copy(dst_hbm.at[pl.ds(base, EHALF)], dst_v)
            lax.fori_loop(0, EHALF // ENBUF, body, 0)

        if h < LOADS_SC1:
            _run()
        else:
            pl.when(c == 0)(_run)
    plsc.subcore_barrier()
    ws = [None] * INIT_CHUNKS
    for k in range(INIT_CHUNKS):
        row0 = s * ROWS_PER_SC_TILE + k * CH
        ws[k] = pltpu.async_copy(
            acc.at[pl.ds(row0, CH)],
            out_hbm.at[pl.ds(c * N_PAD + row0, CH)], sem_w)
    for k in range(INIT_CHUNKS):
        ws[k].wait()


RPT = N_PAD // NW


@functools.partial(
    pl.kernel,
    mesh=_mesh,
    out_type=jax.ShapeDtypeStruct((NW, RPT, D_IN), jnp.float32),
    scratch_types=[
        pltpu.VMEM((RPT,), jnp.int32),
        pltpu.VMEM((RPT, D_IN), jnp.float32),
        pltpu.VMEM_SHARED((N_PAD, D_IN), jnp.float32),
        pltpu.SemaphoreType.DMA,
        pltpu.SemaphoreType.DMA,
    ],
)
def _sc_gather_rows(ls_hbm, rep_hbm, out_hbm, idx_v, rows_v, ls_spm,
                    sem, sem_w):
    c = lax.axis_index("c")
    s = lax.axis_index("s")
    wid = s * NC + c
    for r in range(ROWS_PER_SC_TILE // RPT):
        row0 = s * ROWS_PER_SC_TILE + r * RPT
        pltpu.sync_copy(ls_hbm.at[pl.ds(row0, RPT)], rows_v)
        pltpu.sync_copy(rows_v, ls_spm.at[pl.ds(row0, RPT)])
    pltpu.sync_copy(rep_hbm.at[wid], idx_v)
    plsc.subcore_barrier()
    h1 = pltpu.async_copy(ls_spm.at[idx_v.at[pl.ds(0, CH)]],
                          rows_v.at[pl.ds(0, CH)], sem)
    h2 = pltpu.async_copy(ls_spm.at[idx_v.at[pl.ds(CH, CH)]],
                          rows_v.at[pl.ds(CH, CH)], sem)
    h3 = pltpu.async_copy(ls_spm.at[idx_v.at[pl.ds(2 * CH, RPT - 2 * CH)]],
                          rows_v.at[pl.ds(2 * CH, RPT - 2 * CH)], sem)
    h1.wait()
    h2.wait()
    h3.wait()
    pltpu.sync_copy(rows_v, out_hbm.at[wid])


def _tc_prep(degp, x_pad):
    def body(degp_ref, x_ref, dinv_ref, y0_ref):
        d = degp_ref[0, :, 0:1] + degp_ref[1, :, 0:1] - 1.0
        dinv = lax.rsqrt(jnp.maximum(d, 1.0))
        dinv_ref[...] = dinv
        y0_ref[...] = x_ref[...] * dinv

    return pl.pallas_call(
        body,
        out_shape=(
            jax.ShapeDtypeStruct((N_PAD, 1), jnp.float32),
            jax.ShapeDtypeStruct((N_PAD, D_IN), jnp.float32),
        ),
    )(degp, x_pad)


def _tc_scale2(sp, y0, dinv):
    def body(sp_ref, y0_ref, dinv_ref, y1_ref):
        dv = dinv_ref[...]
        y1_ref[...] = (y0_ref[...] + sp_ref[0] + sp_ref[1]) * (dv * dv)

    return pl.pallas_call(
        body,
        out_shape=jax.ShapeDtypeStruct((N_PAD, D_IN), jnp.float32),
    )(sp, y0, dinv)


def _tc_mlp(sp2, y1, dinv, W1, b1, W2, b2, mult):
    def body(sp_ref, y1_ref, dinv_ref, w1_ref, b1_ref, w2_ref, b2_ref, m_ref,
             ls_ref, h_ref):
        xp = (y1_ref[...] + sp_ref[0] + sp_ref[1]) * dinv_ref[...]
        keys = jnp.round(xp).astype(jnp.int32)
        h_ref[...] = jnp.sum(keys * m_ref[...], axis=1, keepdims=True,
                             dtype=jnp.int32)
        hid = jnp.maximum(
            lax.dot_general(xp, w1_ref[...], (((1,), (1,)), ((), ())),
                            preferred_element_type=jnp.float32)
            + b1_ref[...], 0.0)
        o = lax.dot_general(hid, w2_ref[...], (((1,), (1,)), ((), ())),
                            preferred_element_type=jnp.float32) + b2_ref[...]
        o = o - jnp.max(o, axis=1, keepdims=True)
        ls = o - jnp.log(jnp.sum(jnp.exp(o), axis=1, keepdims=True))
        ls_ref[...] = jnp.concatenate([ls, jnp.zeros_like(ls)], axis=1)

    return pl.pallas_call(
        body,
        out_shape=(
            jax.ShapeDtypeStruct((N_PAD, D_IN), jnp.float32),
            jax.ShapeDtypeStruct((N_PAD, 1), jnp.int32),
        ),
    )(sp2, y1, dinv, W1, b1.reshape(1, D_IN), W2, b2.reshape(1, D_OUT), mult)


REP_BI = 1024
REP_BJ = 512


def _tc_rep(h_col, h_row):
    BIG = 2**30

    def body(hA_ref, hB_ref, rep_ref, best_ref):
        hi = hA_ref[...]
        best_ref[...] = jnp.full((REP_BI, 1), BIG, jnp.int32)
        blk = pl.program_id(0)
        for k in range(N_PAD // REP_BJ):
            @pl.when(k * REP_BJ < (blk + 1) * REP_BI)
            def _():
                hj = hB_ref[:, k * REP_BJ:(k + 1) * REP_BJ]
                eq = hi == hj
                jidx = lax.broadcasted_iota(
                    jnp.int32, (REP_BI, REP_BJ), 1) + jnp.int32(k * REP_BJ)
                cand = jnp.where(eq, jidx, BIG)
                best_ref[...] = jnp.minimum(
                    best_ref[...], jnp.min(cand, axis=1, keepdims=True))

        rep_ref[...] = best_ref[...]

    return pl.pallas_call(
        body,
        grid=(N_PAD // REP_BI,),
        in_specs=[
            pl.BlockSpec((REP_BI, 1), lambda i: (i, 0)),
            pl.BlockSpec((1, N_PAD), lambda i: (0, 0)),
        ],
        out_specs=pl.BlockSpec((REP_BI, 1), lambda i: (i, 0)),
        out_shape=jax.ShapeDtypeStruct((N_PAD, 1), jnp.int32),
        scratch_shapes=[pltpu.VMEM((REP_BI, 1), jnp.int32)],
    )(h_col, h_row)


def kernel(x, edge_index, W1, b1, W2, b2):
    x = x.astype(jnp.float32)
    ei = edge_index.astype(jnp.int32)
    n_extra = E_PAD - N_EDGES_REAL
    src = jnp.concatenate([ei[0], jnp.zeros((n_extra,), jnp.int32)])
    dst = jnp.concatenate([ei[1], jnp.full((n_extra,), JUNK, jnp.int32)])
    src3 = src.reshape(NW, N_IT, CH)
    dst3 = dst.reshape(NW, N_IT, CH)
    x_pad = jnp.pad(x, ((0, N_PAD - N_NODES_REAL), (0, 0)))
    ones128 = jnp.ones((CH, D_IN), jnp.float32)
    zeros128 = jnp.zeros((CH, D_IN), jnp.float32)

    src2h = src.reshape(NCHUNKS, ECH)
    dst2h = dst.reshape(NCHUNKS, ECH)

    degp = _sc_degree(dst3, ones128).reshape(2, N_PAD, D_IN)
    dinv, y0 = _tc_prep(degp, x_pad)
    sp1 = _sc_hop(y0, src2h, dst2h, zeros128).reshape(2, N_PAD, D_IN)
    y1 = _tc_scale2(sp1, y0, dinv)
    sp2 = _sc_hop(y1, src2h, dst2h, zeros128).reshape(2, N_PAD, D_IN)
    ls, h = _tc_mlp(sp2, y1, dinv, W1, b1, W2, b2,
                    jnp.asarray(_MULT_I32, dtype=jnp.int32))
    rep = _tc_rep(h, h.reshape(1, N_PAD))
    out = _sc_gather_rows(ls, rep.reshape(NW, RPT)).reshape(N_PAD, D_IN)
    return out[:N_NODES_REAL, :D_OUT]
